# baseline (speedup 1.0000x reference)
import jax
import jax.numpy as jnp
from jax import lax
from jax.experimental import pallas as pl
from jax.experimental.pallas import tpu as pltpu
from jax.experimental.pallas import tpu_sc as plsc

SEQ = 200
BATCH = 1024
D = 128
ROWS = SEQ * BATCH
NW = 32
ROWS_PER_W = ROWS // NW
CHUNK = 128
NCHUNK = ROWS_PER_W // CHUNK
LANES = 16
VPD = D // LANES


def _embed_body(x_hbm, table_hbm, pe_hbm, out_hbm, idx_v, buf_v, pe_v, sem):
    wid = lax.axis_index("s") * 2 + lax.axis_index("c")
    pltpu.sync_copy(pe_hbm, pe_v)
    base = wid * ROWS_PER_W

    def chunk_body(c, carry):
        start = base + c * CHUNK
        s = start // BATCH
        pltpu.sync_copy(x_hbm.at[pl.ds(start, CHUNK)], idx_v)
        pltpu.async_copy(table_hbm.at[idx_v], buf_v, sem).wait()
        pe_regs = [pe_v[s, pl.ds(LANES * j, LANES)] for j in range(VPD)]

        def row_body(r, rcarry):
            for j in range(VPD):
                plsc.addupdate(buf_v.at[r, pl.ds(LANES * j, LANES)], pe_regs[j])
            return rcarry

        lax.fori_loop(0, CHUNK, row_body, 0)
        pltpu.sync_copy(buf_v, out_hbm.at[pl.ds(start, CHUNK)])
        return carry

    lax.fori_loop(0, NCHUNK, chunk_body, 0)


def kernel(x, table, pe):
    xf = x.reshape(ROWS).astype(jnp.int32)
    pef = pe.reshape(SEQ, D)
    mesh = plsc.VectorSubcoreMesh(core_axis_name="c", subcore_axis_name="s")
    out = pl.kernel(
        _embed_body,
        mesh=mesh,
        out_type=jax.ShapeDtypeStruct((ROWS, D), jnp.float32),
        scratch_types=[
            pltpu.VMEM((CHUNK,), jnp.int32),
            pltpu.VMEM((CHUNK, D), jnp.float32),
            pltpu.VMEM((SEQ, D), jnp.float32),
            pltpu.SemaphoreType.DMA,
        ],
    )(xf, table, pef)
    return out.reshape(SEQ, BATCH, D)


# speedup vs baseline: 1.7284x; 1.7284x over previous
import jax
import jax.numpy as jnp
from jax import lax
from jax.experimental import pallas as pl
from jax.experimental.pallas import tpu as pltpu
from jax.experimental.pallas import tpu_sc as plsc

SEQ = 200
BATCH = 1024
D = 128
ROWS = SEQ * BATCH
NW = 32
ROWS_PER_W = ROWS // NW
CHUNK = 64
NCHUNK = ROWS_PER_W // CHUNK
NBUF = 4
NROUND = NCHUNK // NBUF
LANES = 16
VPD = D // LANES
UNROLL = 4


def _embed_body(x_hbm, table_hbm, pe_hbm, out_hbm, idx_v, pe_v, *ring):
    bufs = ring[:NBUF]
    gsems = ring[NBUF:2 * NBUF]
    osems = ring[2 * NBUF:3 * NBUF]
    wid = lax.axis_index("s") * 2 + lax.axis_index("c")
    base = wid * ROWS_PER_W
    pltpu.sync_copy(pe_hbm, pe_v)
    pltpu.sync_copy(x_hbm.at[wid], idx_v)

    def gather(c, b):
        pltpu.async_copy(table_hbm.at[idx_v.at[c]], bufs[b], gsems[b])

    for b in range(NBUF):
        gather(b, b)

    def add_pe(buf, s):
        pe_regs = [pe_v[s, pl.ds(LANES * j, LANES)] for j in range(VPD)]

        def row_body(r, carry):
            for u in range(UNROLL):
                for j in range(VPD):
                    plsc.addupdate(
                        buf.at[r * UNROLL + u, pl.ds(LANES * j, LANES)],
                        pe_regs[j])
            return carry

        lax.fori_loop(0, CHUNK // UNROLL, row_body, 0)

    def round_body(g, carry):
        for b in range(NBUF):
            c = g * NBUF + b
            start = base + c * CHUNK
            pltpu.make_async_copy(
                table_hbm.at[idx_v.at[c]], bufs[b], gsems[b]).wait()
            add_pe(bufs[b], start // BATCH)
            pltpu.async_copy(bufs[b], out_hbm.at[pl.ds(start, CHUNK)], osems[b])

        @pl.when(g < NROUND - 1)
        def _refill():
            for b in range(NBUF):
                c = (g + 1) * NBUF + b
                pltpu.make_async_copy(
                    bufs[b], out_hbm.at[pl.ds(base, CHUNK)], osems[b]).wait()
                gather(c, b)

        return carry

    lax.fori_loop(0, NROUND, round_body, 0)
    for b in range(NBUF):
        pltpu.make_async_copy(
            bufs[b], out_hbm.at[pl.ds(base, CHUNK)], osems[b]).wait()


def kernel(x, table, pe):
    xf = x.reshape(NW, NCHUNK, CHUNK).astype(jnp.int32)
    pef = pe.reshape(SEQ, D)
    mesh = plsc.VectorSubcoreMesh(core_axis_name="c", subcore_axis_name="s")
    out = pl.kernel(
        _embed_body,
        mesh=mesh,
        out_type=jax.ShapeDtypeStruct((ROWS, D), jnp.float32),
        scratch_types=(
            [pltpu.VMEM((NCHUNK, CHUNK), jnp.int32),
             pltpu.VMEM((SEQ, D), jnp.float32)]
            + [pltpu.VMEM((CHUNK, D), jnp.float32) for _ in range(NBUF)]
            + [pltpu.SemaphoreType.DMA for _ in range(2 * NBUF)]
        ),
    )(xf, table, pef)
    return out.reshape(SEQ, BATCH, D)


# speedup vs baseline: 1.8148x; 1.0500x over previous
import jax
import jax.numpy as jnp
from jax import lax
from jax.experimental import pallas as pl
from jax.experimental.pallas import tpu as pltpu
from jax.experimental.pallas import tpu_sc as plsc

SEQ = 200
BATCH = 1024
D = 128
ROWS = SEQ * BATCH
NW = 32
ROWS_PER_W = ROWS // NW
CHUNK = 128
NCHUNK = ROWS_PER_W // CHUNK
NBUF = 5
NROUND = NCHUNK // NBUF
LANES = 16
VPD = D // LANES
UNROLL = 4


def _embed_body(x_hbm, table_hbm, pe_hbm, out_hbm, idx_v, pe_v, *ring):
    bufs = ring[:NBUF]
    gsems = ring[NBUF:2 * NBUF]
    osems = ring[2 * NBUF:3 * NBUF]
    wid = lax.axis_index("s") * 2 + lax.axis_index("c")
    base = wid * ROWS_PER_W
    pltpu.sync_copy(pe_hbm, pe_v)
    pltpu.sync_copy(x_hbm.at[wid], idx_v)

    def gather(c, b):
        pltpu.async_copy(table_hbm.at[idx_v.at[c]], bufs[b], gsems[b])

    for b in range(NBUF):
        gather(b, b)

    def add_pe(buf, s):
        pe_regs = [pe_v[s, pl.ds(LANES * j, LANES)] for j in range(VPD)]

        def row_body(r, carry):
            for u in range(UNROLL):
                for j in range(VPD):
                    plsc.addupdate(
                        buf.at[r * UNROLL + u, pl.ds(LANES * j, LANES)],
                        pe_regs[j])
            return carry

        lax.fori_loop(0, CHUNK // UNROLL, row_body, 0)

    def round_body(g, carry):
        for b in range(NBUF):
            c = g * NBUF + b
            start = base + c * CHUNK
            pltpu.make_async_copy(
                table_hbm.at[idx_v.at[c]], bufs[b], gsems[b]).wait()
            add_pe(bufs[b], start // BATCH)
            pltpu.async_copy(bufs[b], out_hbm.at[pl.ds(start, CHUNK)], osems[b])

        @pl.when(g < NROUND - 1)
        def _refill():
            for b in range(NBUF):
                c = (g + 1) * NBUF + b
                pltpu.make_async_copy(
                    bufs[b], out_hbm.at[pl.ds(base, CHUNK)], osems[b]).wait()
                gather(c, b)

        return carry

    lax.fori_loop(0, NROUND, round_body, 0)
    for b in range(NBUF):
        pltpu.make_async_copy(
            bufs[b], out_hbm.at[pl.ds(base, CHUNK)], osems[b]).wait()


def kernel(x, table, pe):
    xf = x.reshape(NW, NCHUNK, CHUNK).astype(jnp.int32)
    pef = pe.reshape(SEQ, D)
    mesh = plsc.VectorSubcoreMesh(core_axis_name="c", subcore_axis_name="s")
    out = pl.kernel(
        _embed_body,
        mesh=mesh,
        out_type=jax.ShapeDtypeStruct((ROWS, D), jnp.float32),
        scratch_types=(
            [pltpu.VMEM((NCHUNK, CHUNK), jnp.int32),
             pltpu.VMEM((SEQ, D), jnp.float32)]
            + [pltpu.VMEM((CHUNK, D), jnp.float32) for _ in range(NBUF)]
            + [pltpu.SemaphoreType.DMA for _ in range(2 * NBUF)]
        ),
    )(xf, table, pef)
    return out.reshape(SEQ, BATCH, D)


# speedup vs baseline: 1.9158x; 1.0556x over previous
import jax
import jax.numpy as jnp
from jax import lax
from jax.experimental import pallas as pl
from jax.experimental.pallas import tpu as pltpu
from jax.experimental.pallas import tpu_sc as plsc

SEQ = 200
BATCH = 1024
D = 128
ROWS = SEQ * BATCH
NW = 32
ROWS_PER_W = ROWS // NW
CHUNK = 128
NCHUNK = ROWS_PER_W // CHUNK
NBUF = 5
NROUND = NCHUNK // NBUF
PE_ROWS = 16
LANES = 16
VPD = D // LANES
UNROLL = 4


def _embed_body(x_hbm, table_hbm, pe_hbm, out_hbm, idx_v, pe_v, *ring):
    bufs = ring[:NBUF]
    gsems = ring[NBUF:2 * NBUF]
    osems = ring[2 * NBUF:3 * NBUF]
    wid = lax.axis_index("s") * 2 + lax.axis_index("c")
    base = wid * ROWS_PER_W
    s0 = lax.min((base // BATCH) // 8 * 8, SEQ - PE_ROWS)
    pltpu.sync_copy(pe_hbm.at[pl.ds(s0, PE_ROWS)], pe_v)
    pltpu.sync_copy(x_hbm.at[wid], idx_v)

    def gather(c, b):
        pltpu.async_copy(table_hbm.at[idx_v.at[c]], bufs[b], gsems[b])

    for b in range(NBUF):
        gather(b, b)

    def add_pe(buf, s):
        pe_regs = [pe_v[s, pl.ds(LANES * j, LANES)] for j in range(VPD)]

        def row_body(r, carry):
            for u in range(UNROLL):
                for j in range(VPD):
                    plsc.addupdate(
                        buf.at[r * UNROLL + u, pl.ds(LANES * j, LANES)],
                        pe_regs[j])
            return carry

        lax.fori_loop(0, CHUNK // UNROLL, row_body, 0)

    def round_body(g, carry):
        for b in range(NBUF):
            c = g * NBUF + b
            start = base + c * CHUNK
            pltpu.make_async_copy(
                table_hbm.at[idx_v.at[c]], bufs[b], gsems[b]).wait()
            add_pe(bufs[b], start // BATCH - s0)
            pltpu.async_copy(bufs[b], out_hbm.at[pl.ds(start, CHUNK)], osems[b])

        @pl.when(g < NROUND - 1)
        def _refill():
            for b in range(NBUF):
                c = (g + 1) * NBUF + b
                pltpu.make_async_copy(
                    bufs[b], out_hbm.at[pl.ds(base, CHUNK)], osems[b]).wait()
                gather(c, b)

        return carry

    lax.fori_loop(0, NROUND, round_body, 0)
    for b in range(NBUF):
        pltpu.make_async_copy(
            bufs[b], out_hbm.at[pl.ds(base, CHUNK)], osems[b]).wait()


def kernel(x, table, pe):
    xf = x.reshape(NW, NCHUNK, CHUNK).astype(jnp.int32)
    pef = pe.reshape(SEQ, D)
    mesh = plsc.VectorSubcoreMesh(core_axis_name="c", subcore_axis_name="s")
    out = pl.kernel(
        _embed_body,
        mesh=mesh,
        out_type=jax.ShapeDtypeStruct((ROWS, D), jnp.float32),
        scratch_types=(
            [pltpu.VMEM((NCHUNK, CHUNK), jnp.int32),
             pltpu.VMEM((PE_ROWS, D), jnp.float32)]
            + [pltpu.VMEM((CHUNK, D), jnp.float32) for _ in range(NBUF)]
            + [pltpu.SemaphoreType.DMA for _ in range(2 * NBUF)]
        ),
    )(xf, table, pef)
    return out.reshape(SEQ, BATCH, D)
